# Initial kernel scaffold; baseline (speedup 1.0000x reference)
import jax
import jax.numpy as jnp
from jax.experimental import pallas as pl


def kernel(u, v, emb, W1, b1, W2, b2, W3, b3):
    raise NotImplementedError("write your pallas kernel here")


# speedup vs baseline: 1.5981x; 1.5981x over previous
import functools

import jax
import jax.numpy as jnp
from jax import lax
from jax.experimental import pallas as pl
from jax.experimental.pallas import tpu as pltpu
from jax.experimental.pallas import tpu_sc as plsc

_NC = 2
_NS = 16


def _gather_pair(emb, u, v):
    B = u.shape[0]
    D = emb.shape[1]
    nw = _NC * _NS
    bpw = B // nw
    mesh = plsc.VectorSubcoreMesh(core_axis_name="c", subcore_axis_name="s")

    @functools.partial(
        pl.kernel,
        mesh=mesh,
        compiler_params=pltpu.CompilerParams(use_tc_tiling_on_sc=False),
        out_type=[
            jax.ShapeDtypeStruct((B, D), jnp.float32),
            jax.ShapeDtypeStruct((B, D), jnp.float32),
        ],
        scratch_types=[
            pltpu.VMEM((bpw,), jnp.int32),
            pltpu.VMEM((bpw, D), jnp.float32),
            pltpu.VMEM((bpw,), jnp.int32),
            pltpu.VMEM((bpw, D), jnp.float32),
            pltpu.SemaphoreType.DMA,
            pltpu.SemaphoreType.DMA,
        ],
    )
    def gather_k(emb_hbm, u_hbm, v_hbm, ue_hbm, ve_hbm,
                 ui, ur, vi, vr, su, sv):
        wid = lax.axis_index("s") * _NC + lax.axis_index("c")
        base = wid * bpw
        pltpu.sync_copy(u_hbm.at[pl.ds(base, bpw)], ui)
        pltpu.sync_copy(v_hbm.at[pl.ds(base, bpw)], vi)
        cu = pltpu.async_copy(emb_hbm.at[ui], ur, su)
        cv = pltpu.async_copy(emb_hbm.at[vi], vr, sv)
        cu.wait()
        pltpu.sync_copy(ur, ue_hbm.at[pl.ds(base, bpw)])
        cv.wait()
        pltpu.sync_copy(vr, ve_hbm.at[pl.ds(base, bpw)])

    return gather_k(emb, u, v)


def _mlp(ue, ve, w1u, w1v, b1, w2, b2, w3, b3):
    B, D = ue.shape
    blk = 2048

    def body(ue_ref, ve_ref, w1u_ref, w1v_ref, b1_ref, w2_ref, b2_ref,
             w3_ref, b3_ref, o_ref):
        x1 = jnp.dot(ue_ref[...], w1u_ref[...], preferred_element_type=jnp.float32)
        x2 = jnp.dot(ve_ref[...], w1v_ref[...], preferred_element_type=jnp.float32)
        h = jnp.maximum(x1 + x2 + b1_ref[...], 0.0)
        h = jnp.maximum(
            jnp.dot(h, w2_ref[...], preferred_element_type=jnp.float32) + b2_ref[...], 0.0)
        o_ref[...] = jnp.dot(h, w3_ref[...], preferred_element_type=jnp.float32) + b3_ref[...]

    return pl.pallas_call(
        body,
        grid=(B // blk,),
        in_specs=[
            pl.BlockSpec((blk, D), lambda i: (i, 0)),
            pl.BlockSpec((blk, D), lambda i: (i, 0)),
            pl.BlockSpec((D, 128), lambda i: (0, 0)),
            pl.BlockSpec((D, 128), lambda i: (0, 0)),
            pl.BlockSpec((1, 128), lambda i: (0, 0)),
            pl.BlockSpec((128, 64), lambda i: (0, 0)),
            pl.BlockSpec((1, 64), lambda i: (0, 0)),
            pl.BlockSpec((64, 1), lambda i: (0, 0)),
            pl.BlockSpec((1, 1), lambda i: (0, 0)),
        ],
        out_specs=pl.BlockSpec((blk, 1), lambda i: (i, 0)),
        out_shape=jax.ShapeDtypeStruct((B, 1), jnp.float32),
    )(ue, ve, w1u, w1v, b1, w2, b2, w3, b3)


def kernel(u, v, emb, W1, b1, W2, b2, W3, b3):
    u = u.astype(jnp.int32)
    v = v.astype(jnp.int32)
    D = emb.shape[1]
    ue, ve = _gather_pair(emb, u, v)
    return _mlp(
        ue, ve,
        W1[:, :D].T, W1[:, D:].T, b1.reshape(1, -1),
        W2.T, b2.reshape(1, -1),
        W3.T, b3.reshape(1, 1),
    )


# speedup vs baseline: 3.2496x; 2.0334x over previous
import functools

import jax
import jax.numpy as jnp
from jax import lax
from jax.experimental import pallas as pl
from jax.experimental.pallas import tpu as pltpu
from jax.experimental.pallas import tpu_sc as plsc

_NC = 2
_NS = 16

_CHUNK = 8192


def _gather_transposed(emb_t, u, v):
    D, V = emb_t.shape
    B = u.shape[0]

    mesh = plsc.VectorSubcoreMesh(core_axis_name="c", subcore_axis_name="s")

    @functools.partial(
        pl.kernel,
        mesh=mesh,
        compiler_params=pltpu.CompilerParams(needs_layout_passes=False),
        out_type=jax.ShapeDtypeStruct((2 * D, B), jnp.float32),
        scratch_types=[
            pltpu.VMEM((V,), jnp.float32),
            pltpu.VMEM((_CHUNK,), jnp.int32),
            pltpu.VMEM((_CHUNK,), jnp.float32),
        ],
    )
    def gather_k(emb_hbm, u_hbm, v_hbm, xt_hbm, feat, idxb, outb):
        wid = lax.axis_index("s") * _NC + lax.axis_index("c")
        pltpu.sync_copy(emb_hbm.at[wid], feat)

        def gather_chunk(idx_hbm, out_row, c):
            base = c * _CHUNK
            pltpu.sync_copy(idx_hbm.at[pl.ds(base, _CHUNK)], idxb)

            def body(i, _):
                iv = idxb[pl.ds(i * 16, 16)]
                outb[pl.ds(i * 16, 16)] = plsc.load_gather(feat, [iv])
                return 0

            lax.fori_loop(0, _CHUNK // 16, body, 0)
            pltpu.sync_copy(outb, xt_hbm.at[out_row, pl.ds(base, _CHUNK)])

        for c in range(B // _CHUNK):
            gather_chunk(u_hbm, wid, c)
        for c in range(B // _CHUNK):
            gather_chunk(v_hbm, wid + D, c)

    return gather_k(emb_t, u, v)


def _mlp_t(xt, w1, b1, w2, b2, w3, b3):
    D2, B = xt.shape
    blk = 4096
    dot = functools.partial(
        lax.dot_general, preferred_element_type=jnp.float32)
    dims = (((1,), (0,)), ((), ()))

    def body(xt_ref, w1_ref, b1_ref, w2_ref, b2_ref, w3_ref, b3_ref, o_ref):
        h = jnp.maximum(dot(w1_ref[...], xt_ref[...], dims) + b1_ref[...], 0.0)
        h = jnp.maximum(dot(w2_ref[...], h, dims) + b2_ref[...], 0.0)
        o_ref[...] = dot(w3_ref[...], h, dims) + b3_ref[...]

    return pl.pallas_call(
        body,
        grid=(B // blk,),
        in_specs=[
            pl.BlockSpec((D2, blk), lambda i: (0, i)),
            pl.BlockSpec((128, D2), lambda i: (0, 0)),
            pl.BlockSpec((128, 1), lambda i: (0, 0)),
            pl.BlockSpec((64, 128), lambda i: (0, 0)),
            pl.BlockSpec((64, 1), lambda i: (0, 0)),
            pl.BlockSpec((1, 64), lambda i: (0, 0)),
            pl.BlockSpec((1, 1), lambda i: (0, 0)),
        ],
        out_specs=pl.BlockSpec((1, blk), lambda i: (0, i)),
        out_shape=jax.ShapeDtypeStruct((1, B), jnp.float32),
    )(xt, w1, b1, w2, b2, w3, b3)


def kernel(u, v, emb, W1, b1, W2, b2, W3, b3):
    u = u.astype(jnp.int32)
    v = v.astype(jnp.int32)
    xt = _gather_transposed(emb.T, u, v)
    out_t = _mlp_t(xt, W1, b1.reshape(-1, 1), W2, b2.reshape(-1, 1),
                   W3, b3.reshape(-1, 1))
    return out_t.reshape(-1, 1)


# speedup vs baseline: 3.9339x; 1.2106x over previous
import functools

import jax
import jax.numpy as jnp
from jax import lax
from jax.experimental import pallas as pl
from jax.experimental.pallas import tpu as pltpu
from jax.experimental.pallas import tpu_sc as plsc

_NC = 2
_NS = 16

_CHUNK = 8192


def _gather_transposed(emb_t, u, v):
    D, V = emb_t.shape
    B = u.shape[0]

    mesh = plsc.VectorSubcoreMesh(core_axis_name="c", subcore_axis_name="s")

    @functools.partial(
        pl.kernel,
        mesh=mesh,
        compiler_params=pltpu.CompilerParams(needs_layout_passes=False),
        out_type=jax.ShapeDtypeStruct((2 * D, B), jnp.float32),
        scratch_types=[
            pltpu.VMEM((V,), jnp.float32),
            pltpu.VMEM((_CHUNK,), jnp.int32),
            pltpu.VMEM((_CHUNK,), jnp.float32),
        ],
    )
    def gather_k(emb_hbm, u_hbm, v_hbm, xt_hbm, feat, idxb, outb):
        wid = lax.axis_index("s") * _NC + lax.axis_index("c")
        pltpu.sync_copy(emb_hbm.at[wid], feat)

        def gather_chunk(idx_hbm, out_row, c):
            base = c * _CHUNK
            pltpu.sync_copy(idx_hbm.at[pl.ds(base, _CHUNK)], idxb)

            @plsc.parallel_loop(0, _CHUNK // 16, unroll=8)
            def _(i):
                iv = idxb[pl.ds(i * 16, 16)]
                outb[pl.ds(i * 16, 16)] = plsc.load_gather(feat, [iv])
            pltpu.sync_copy(outb, xt_hbm.at[out_row, pl.ds(base, _CHUNK)])

        for c in range(B // _CHUNK):
            gather_chunk(u_hbm, wid, c)
        for c in range(B // _CHUNK):
            gather_chunk(v_hbm, wid + D, c)

    return gather_k(emb_t, u, v)


def _mlp_t(xt, w1, b1, w2, b2, w3, b3):
    D2, B = xt.shape
    blk = 4096
    dot = functools.partial(
        lax.dot_general, preferred_element_type=jnp.float32)
    dims = (((1,), (0,)), ((), ()))

    def body(xt_ref, w1_ref, b1_ref, w2_ref, b2_ref, w3_ref, b3_ref, o_ref):
        h = jnp.maximum(dot(w1_ref[...], xt_ref[...], dims) + b1_ref[...], 0.0)
        h = jnp.maximum(dot(w2_ref[...], h, dims) + b2_ref[...], 0.0)
        o_ref[...] = dot(w3_ref[...], h, dims) + b3_ref[...]

    return pl.pallas_call(
        body,
        grid=(B // blk,),
        in_specs=[
            pl.BlockSpec((D2, blk), lambda i: (0, i)),
            pl.BlockSpec((128, D2), lambda i: (0, 0)),
            pl.BlockSpec((128, 1), lambda i: (0, 0)),
            pl.BlockSpec((64, 128), lambda i: (0, 0)),
            pl.BlockSpec((64, 1), lambda i: (0, 0)),
            pl.BlockSpec((1, 64), lambda i: (0, 0)),
            pl.BlockSpec((1, 1), lambda i: (0, 0)),
        ],
        out_specs=pl.BlockSpec((1, blk), lambda i: (0, i)),
        out_shape=jax.ShapeDtypeStruct((1, B), jnp.float32),
    )(xt, w1, b1, w2, b2, w3, b3)


def kernel(u, v, emb, W1, b1, W2, b2, W3, b3):
    u = u.astype(jnp.int32)
    v = v.astype(jnp.int32)
    xt = _gather_transposed(emb.T, u, v)
    out_t = _mlp_t(xt, W1, b1.reshape(-1, 1), W2, b2.reshape(-1, 1),
                   W3, b3.reshape(-1, 1))
    return out_t.reshape(-1, 1)
